# baseline (speedup 1.0000x reference)
import math

import jax
import jax.numpy as jnp
import numpy as np
from jax import lax
from jax.experimental import pallas as pl
from jax.experimental.pallas import tpu as pltpu
from jax.experimental.pallas import tpu_sc as plsc

VOCAB = 1000000
D = 64
B = 1024
L = 200

_N = B * L
_SHARD = 32768
_SS_ROWS = 512
_NSS_FULL = _SHARD // _SS_ROWS
_CHUNK = 2048
_NCHUNK = _N // _CHUNK
_REG_CAP = 2048
_OCT_CAP = 1152
_SS_CAP = 256
_DUMP = _N


def _pos_encoding_np(max_len, d):
    pos = np.arange(max_len)[:, None].astype(np.float32)
    i = np.arange(0, d, 2).astype(np.float32)
    div = np.exp(-math.log(10000.0) * i / float(d))
    pe = np.zeros((max_len, d), dtype=np.float32)
    pe[:, 0::2] = np.sin(pos * div)
    pe[:, 1::2] = np.cos(pos * div)
    return pe


_PE_FLAT = _pos_encoding_np(L, D).reshape(-1)


def _sweep_body(tT_hbm, xi_hbm, pe_hbm, tailT_hbm, out_hbm,
                win0, win1, chunk0, chunk1, pn_v, pr_v, on_v, or_v,
                ssn_v, ssr_v, pe_v, line0, line1, nst0, nst1, scnt, ocnt_s, rend_s,
                wsem0, wsem1, csem0, csem1, lsem0, lsem1):
    wid = lax.axis_index("s") * 2 + lax.axis_index("c")
    lo = wid * _SHARD
    iota = lax.broadcasted_iota(jnp.int32, (16,), 0)
    nss = jnp.where(wid < 30, _NSS_FULL, jnp.where(wid == 30, 34, 0))

    pltpu.sync_copy(pe_hbm, pe_v)

    wins = ((win0, wsem0), (win1, wsem1))
    chunks = ((chunk0, csem0), (chunk1, csem1))
    lines = ((line0, nst0, lsem0), (line1, nst1, lsem1))

    def start_win(ss, buf, sem):
        tail = (wid == 30) & (ss == 33)

        @pl.when((ss < nss) & jnp.logical_not(tail))
        def _full():
            base = lo + ss * _SS_ROWS
            for w in range(4):
                pltpu.async_copy(
                    tT_hbm.at[:, pl.ds(base + w * 128, 128)],
                    buf.at[pl.ds(w * 64, 64)], sem)

        @pl.when(tail)
        def _tail():
            pltpu.async_copy(tailT_hbm, buf.at[pl.ds(0, 64)], sem)

    def wait_win(ss, buf, sem):
        tail = (wid == 30) & (ss == 33)

        @pl.when((ss < nss) & jnp.logical_not(tail))
        def _full():
            pltpu.make_async_copy(tT_hbm.at[:, pl.ds(0, 512)], buf, sem).wait()

        @pl.when(tail)
        def _tail():
            pltpu.make_async_copy(
                tT_hbm.at[:, pl.ds(0, 128)],
                buf.at[pl.ds(0, 64)], sem).wait()

    def popcnt(m):
        res = plsc.all_reduce_population_count(m)
        return res if res.ndim == 0 else res[0]

    def vcompress(ref_r, ref_n, rv, nv, m, base_v):
        pre = plsc.cumsum(m.astype(jnp.int32))
        pos = base_v + pre - 1
        plsc.store_scatter(ref_r, [pos], rv, mask=m)
        plsc.store_scatter(ref_n, [pos], nv, mask=m)
        return base_v + plsc.all_reduce_population_count(m)

    c0 = lax.rem(wid * 7, _NCHUNK)

    def chunk_slice(k):
        ci = c0 + k
        ci = jnp.where(ci >= _NCHUNK, ci - _NCHUNK, ci)
        return ci * _CHUNK

    for b in range(2):
        pltpu.async_copy(xi_hbm.at[pl.ds(chunk_slice(b), _CHUNK)],
                         chunks[b][0], chunks[b][1])

    _G = _CHUNK // 16

    def scan_chunk(k, cbuf, csem, ptrs):
        pltpu.make_async_copy(xi_hbm.at[pl.ds(0, _CHUNK)], cbuf, csem).wait()
        nbase = chunk_slice(k)

        @pl.loop(0, _G // 4, init_carry=ptrs)
        def _grp(g, ps):
            out = []
            for q in range(4):
                gq = g + q * (_G // 4)
                xv = cbuf[pl.ds(gq * 16, 16)]
                m = (xv >= lo) & (xv < lo + _SHARD)
                nv = nbase + gq * 16 + iota
                out.append(vcompress(pr_v, pn_v, xv, nv, m, ps[q]))
            return tuple(out)

        ptrs = _grp

        @pl.when(k + 2 < _NCHUNK)
        def _next():
            pltpu.async_copy(xi_hbm.at[pl.ds(chunk_slice(k + 2), _CHUNK)],
                             cbuf, csem)

        return ptrs

    @pl.loop(0, _NCHUNK // 2,
             init_carry=tuple(jnp.full((16,), q * _REG_CAP, jnp.int32)
                              for q in range(4)))
    def _scan(h, ptrs):
        for b in range(2):
            ptrs = scan_chunk(h * 2 + b, chunks[b][0], chunks[b][1], ptrs)
        return ptrs

    rends = _scan

    for q in range(4):
        rend_s[q] = rends[q][0] - q * _REG_CAP

    @pl.loop(0, 4, init_carry=tuple(jnp.full((16,), oi * _OCT_CAP, jnp.int32)
                                    for oi in range(8)))
    def _regions(q, optrs):
        rbase = q * _REG_CAP
        rcnt = rend_s[q]

        @pl.loop(0, (rcnt + 15) >> 4, init_carry=optrs)
        def _bkt(g, ops):
            rv = pr_v[pl.ds(rbase + g * 16, 16)]
            nv = pn_v[pl.ds(rbase + g * 16, 16)]
            valid = (g * 16 + iota) < rcnt
            ob = (rv - lo) >> 12
            out = []
            for oi in range(8):
                m = valid & (ob == oi)
                out.append(vcompress(or_v, on_v, rv, nv, m, ops[oi]))
            return tuple(out)

        return _bkt

    for oi in range(8):
        ocnt_s[oi] = _regions[oi][0] - oi * _OCT_CAP

    dump16 = jnp.full((16,), _DUMP, jnp.int32)
    for lbuf, nst, lsem in lines:
        nst[pl.ds(0, 16)] = dump16
        nst[pl.ds(16, 16)] = dump16
        pltpu.async_copy(lbuf, out_hbm.at[nst], lsem)

    for b in range(2):
        start_win(jnp.int32(b), wins[b][0], wins[b][1])

    def fill_ss_lists(o):
        obase_r = lo + o * (8 * _SS_ROWS)
        ostart = o * _OCT_CAP
        ocnt = ocnt_s[o]

        @pl.loop(0, (ocnt + 15) >> 4,
                 init_carry=tuple(jnp.full((16,), s * _SS_CAP, jnp.int32)
                                  for s in range(8)))
        def _fill(g, sps):
            rv = or_v[pl.ds(ostart + g * 16, 16)]
            nv = on_v[pl.ds(ostart + g * 16, 16)]
            valid = (g * 16 + iota) < ocnt
            sb = (rv - obase_r) >> 9
            out = []
            for si in range(8):
                m = valid & (sb == si)
                out.append(vcompress(ssr_v, ssn_v, rv, nv, m, sps[si]))
            return tuple(out)

        sends = _fill
        for si in range(8):
            scnt[si] = sends[si][0] - si * _SS_CAP

    def do_subshard(ss, si, wbuf, wsem):
        wait_win(ss, wbuf, wsem)
        ss_base = lo + ss * _SS_ROWS
        cnt = scnt[si]
        sbase = si * _SS_CAP
        nbatch = (cnt + 31) >> 5

        @pl.loop(0, (nbatch + 1) >> 1)
        def _batchpair(tp):
            for b2, (lbuf, nst, lsem) in enumerate(lines):
                t = tp * 2 + b2

                @pl.when(t < nbatch)
                def _do():
                    pltpu.make_async_copy(out_hbm.at[pl.ds(0, 32)], lbuf,
                                          lsem).wait()
                    for gg in range(2):
                        j0 = t * 32 + gg * 16
                        rv = ssr_v[pl.ds(sbase + j0, 16)]
                        nv = ssn_v[pl.ds(sbase + j0, 16)]
                        vm = (j0 + iota) < cnt
                        rloc = (rv - ss_base) & (_SS_ROWS - 1)
                        lane = rloc & 127
                        wrow = (rloc >> 7) << 6
                        nv_s = jnp.where(vm, nv, 0)
                        pos64 = lax.rem(nv_s, L) * D
                        nst[pl.ds(gg * 16, 16)] = jnp.where(vm, nv, dump16)

                        @pl.loop(0, D, unroll=8)
                        def _feat(c):
                            tok = plsc.load_gather(wbuf, [wrow + c, lane])
                            pv = plsc.load_gather(pe_v, [pos64 + c])
                            plsc.store_scatter(
                                lbuf, [gg * 16 + iota, iota * 0 + c],
                                tok + pv)
                    pltpu.async_copy(lbuf, out_hbm.at[nst], lsem)

        start_win(ss + 2, wbuf, wsem)

    @pl.loop(0, 8)
    def _octloop(o):
        fill_ss_lists(o)

        @pl.loop(0, 4)
        def _sspair(sp):
            for b in range(2):
                si = sp * 2 + b
                ss = o * 8 + si

                @pl.when(ss < nss)
                def _run():
                    do_subshard(ss, si, wins[b][0], wins[b][1])

    for lbuf, nst, lsem in lines:
        pltpu.make_async_copy(out_hbm.at[pl.ds(0, 32)], lbuf, lsem).wait()


@jax.jit
def _sweep(tT, xi, pe, tailT):
    f = pl.kernel(
        _sweep_body,
        mesh=plsc.VectorSubcoreMesh(core_axis_name="c", subcore_axis_name="s"),
        out_type=jax.ShapeDtypeStruct((_N + 1, 128), jnp.float32),
        scratch_types=[
            pltpu.VMEM((_SS_ROWS // 2, 128), jnp.float32),
            pltpu.VMEM((_SS_ROWS // 2, 128), jnp.float32),
            pltpu.VMEM((_CHUNK,), jnp.int32),
            pltpu.VMEM((_CHUNK,), jnp.int32),
            pltpu.VMEM((4 * _REG_CAP,), jnp.int32),
            pltpu.VMEM((4 * _REG_CAP,), jnp.int32),
            pltpu.VMEM((8 * _OCT_CAP,), jnp.int32),
            pltpu.VMEM((8 * _OCT_CAP,), jnp.int32),
            pltpu.VMEM((8 * _SS_CAP,), jnp.int32),
            pltpu.VMEM((8 * _SS_CAP,), jnp.int32),
            pltpu.VMEM((L * D,), jnp.float32),
            pltpu.VMEM((32, 128), jnp.float32),
            pltpu.VMEM((32, 128), jnp.float32),
            pltpu.VMEM((32,), jnp.int32),
            pltpu.VMEM((32,), jnp.int32),
            pltpu.SMEM((8,), jnp.int32),
            pltpu.SMEM((8,), jnp.int32),
            pltpu.SMEM((4,), jnp.int32),
            pltpu.SemaphoreType.DMA,
            pltpu.SemaphoreType.DMA,
            pltpu.SemaphoreType.DMA,
            pltpu.SemaphoreType.DMA,
            pltpu.SemaphoreType.DMA,
            pltpu.SemaphoreType.DMA,
        ],
        compiler_params=pltpu.CompilerParams(
            use_tc_tiling_on_sc=True, needs_layout_passes=False
        ),
    )
    return f(tT, xi, pe, tailT)


def kernel(x, table):
    tT = table.T
    xi = x.astype(jnp.int32).reshape(_N)
    tailT = jnp.pad(tT[:, VOCAB - 64:], ((0, 0), (0, 64)))
    padded = _sweep(tT, xi, jnp.asarray(_PE_FLAT), tailT)
    return padded[:_N, :64].reshape(B, L, D)


# speedup vs baseline: 1.5164x; 1.5164x over previous
import math

import jax
import jax.numpy as jnp
import numpy as np
from jax import lax
from jax.experimental import pallas as pl
from jax.experimental.pallas import tpu as pltpu
from jax.experimental.pallas import tpu_sc as plsc

VOCAB = 1000000
D = 64
B = 1024
L = 200

_N = B * L
_SHARD = 32768
_SS_ROWS = 512
_NSS_FULL = _SHARD // _SS_ROWS
_CHUNK = 2048
_NCHUNK = _N // _CHUNK
_REG_CAP = 1920
_OCT_CAP = 1056
_SS_CAP = 232
_DUMP = _N


def _pos_encoding_np(max_len, d):
    pos = np.arange(max_len)[:, None].astype(np.float32)
    i = np.arange(0, d, 2).astype(np.float32)
    div = np.exp(-math.log(10000.0) * i / float(d))
    pe = np.zeros((max_len, d), dtype=np.float32)
    pe[:, 0::2] = np.sin(pos * div)
    pe[:, 1::2] = np.cos(pos * div)
    return pe


_PE_FLAT = _pos_encoding_np(L, D).reshape(-1)


def _sweep_body(tT_hbm, xi_hbm, pe_hbm, tailT_hbm, out_hbm,
                win0, win1, chunk0, chunk1, pn_v, pr_v, on_v, or_v,
                ssn_v, ssr_v, pe_v, line0, line1, line2, line3, line4,
                nst0, nst1, nst2, nst3, nst4, scnt, ocnt_s, rend_s,
                wsem0, wsem1, csem0, csem1,
                lsem0, lsem1, lsem2, lsem3, lsem4):
    wid = lax.axis_index("s") * 2 + lax.axis_index("c")
    lo = wid * _SHARD
    iota = lax.broadcasted_iota(jnp.int32, (16,), 0)
    nss = jnp.where(wid < 30, _NSS_FULL, jnp.where(wid == 30, 34, 0))

    pltpu.sync_copy(pe_hbm, pe_v)

    wins = ((win0, wsem0), (win1, wsem1))
    chunks = ((chunk0, csem0), (chunk1, csem1))
    lines = ((line0, nst0, lsem0), (line1, nst1, lsem1),
             (line2, nst2, lsem2), (line3, nst3, lsem3),
             (line4, nst4, lsem4))

    def start_win(ss, buf, sem):
        tail = (wid == 30) & (ss == 33)

        @pl.when((ss < nss) & jnp.logical_not(tail))
        def _full():
            base = lo + ss * _SS_ROWS
            for w in range(4):
                pltpu.async_copy(
                    tT_hbm.at[:, pl.ds(base + w * 128, 128)],
                    buf.at[pl.ds(w * 64, 64)], sem)

        @pl.when(tail)
        def _tail():
            pltpu.async_copy(tailT_hbm, buf.at[pl.ds(0, 64)], sem)

    def wait_win(ss, buf, sem):
        tail = (wid == 30) & (ss == 33)

        @pl.when((ss < nss) & jnp.logical_not(tail))
        def _full():
            pltpu.make_async_copy(tT_hbm.at[:, pl.ds(0, 512)], buf, sem).wait()

        @pl.when(tail)
        def _tail():
            pltpu.make_async_copy(
                tT_hbm.at[:, pl.ds(0, 128)],
                buf.at[pl.ds(0, 64)], sem).wait()

    def popcnt(m):
        res = plsc.all_reduce_population_count(m)
        return res if res.ndim == 0 else res[0]

    def vcompress(ref_r, ref_n, rv, nv, m, base_v):
        pre = plsc.cumsum(m.astype(jnp.int32))
        pos = base_v + pre - 1
        plsc.store_scatter(ref_r, [pos], rv, mask=m)
        plsc.store_scatter(ref_n, [pos], nv, mask=m)
        return base_v + plsc.all_reduce_population_count(m)

    c0 = lax.rem(wid * 7, _NCHUNK)

    def chunk_slice(k):
        ci = c0 + k
        ci = jnp.where(ci >= _NCHUNK, ci - _NCHUNK, ci)
        return ci * _CHUNK

    for b in range(2):
        pltpu.async_copy(xi_hbm.at[pl.ds(chunk_slice(b), _CHUNK)],
                         chunks[b][0], chunks[b][1])

    _G = _CHUNK // 16

    def scan_chunk(k, cbuf, csem, ptrs):
        pltpu.make_async_copy(xi_hbm.at[pl.ds(0, _CHUNK)], cbuf, csem).wait()
        nbase = chunk_slice(k)

        @pl.loop(0, _G // 4, init_carry=ptrs)
        def _grp(g, ps):
            out = []
            for q in range(4):
                gq = g + q * (_G // 4)
                xv = cbuf[pl.ds(gq * 16, 16)]
                m = (xv >= lo) & (xv < lo + _SHARD)
                nv = nbase + gq * 16 + iota
                out.append(vcompress(pr_v, pn_v, xv, nv, m, ps[q]))
            return tuple(out)

        ptrs = _grp

        @pl.when(k + 2 < _NCHUNK)
        def _next():
            pltpu.async_copy(xi_hbm.at[pl.ds(chunk_slice(k + 2), _CHUNK)],
                             cbuf, csem)

        return ptrs

    @pl.loop(0, _NCHUNK // 2,
             init_carry=tuple(jnp.full((16,), q * _REG_CAP, jnp.int32)
                              for q in range(4)))
    def _scan(h, ptrs):
        for b in range(2):
            ptrs = scan_chunk(h * 2 + b, chunks[b][0], chunks[b][1], ptrs)
        return ptrs

    rends = _scan

    for q in range(4):
        rend_s[q] = rends[q][0] - q * _REG_CAP

    @pl.loop(0, 4, init_carry=tuple(jnp.full((16,), oi * _OCT_CAP, jnp.int32)
                                    for oi in range(8)))
    def _regions(q, optrs):
        rbase = q * _REG_CAP
        rcnt = rend_s[q]

        @pl.loop(0, (rcnt + 15) >> 4, init_carry=optrs)
        def _bkt(g, ops):
            rv = pr_v[pl.ds(rbase + g * 16, 16)]
            nv = pn_v[pl.ds(rbase + g * 16, 16)]
            valid = (g * 16 + iota) < rcnt
            ob = (rv - lo) >> 12
            out = []
            for oi in range(8):
                m = valid & (ob == oi)
                out.append(vcompress(or_v, on_v, rv, nv, m, ops[oi]))
            return tuple(out)

        return _bkt

    for oi in range(8):
        ocnt_s[oi] = _regions[oi][0] - oi * _OCT_CAP

    dump16 = jnp.full((16,), _DUMP, jnp.int32)
    for lbuf, nst, lsem in lines:
        nst[pl.ds(0, 16)] = dump16
        pltpu.async_copy(lbuf, out_hbm.at[nst], lsem)

    for b in range(2):
        start_win(jnp.int32(b), wins[b][0], wins[b][1])

    def fill_ss_lists(o):
        obase_r = lo + o * (8 * _SS_ROWS)
        ostart = o * _OCT_CAP
        ocnt = ocnt_s[o]

        @pl.loop(0, (ocnt + 15) >> 4,
                 init_carry=tuple(jnp.full((16,), s * _SS_CAP, jnp.int32)
                                  for s in range(8)))
        def _fill(g, sps):
            rv = or_v[pl.ds(ostart + g * 16, 16)]
            nv = on_v[pl.ds(ostart + g * 16, 16)]
            valid = (g * 16 + iota) < ocnt
            sb = (rv - obase_r) >> 9
            out = []
            for si in range(8):
                m = valid & (sb == si)
                out.append(vcompress(ssr_v, ssn_v, rv, nv, m, sps[si]))
            return tuple(out)

        sends = _fill
        for si in range(8):
            scnt[si] = sends[si][0] - si * _SS_CAP

    def do_subshard(ss, si, wbuf, wsem):
        wait_win(ss, wbuf, wsem)
        ss_base = lo + ss * _SS_ROWS
        cnt = scnt[si]
        sbase = si * _SS_CAP
        nbatch = (cnt + 15) >> 4

        @pl.loop(0, 4)
        def _batchpair(tp):
            for b2, (lbuf, nst, lsem) in enumerate(lines):
                t = tp * 5 + b2

                @pl.when(t < nbatch)
                def _do():
                    pltpu.make_async_copy(out_hbm.at[pl.ds(0, 16)], lbuf,
                                          lsem).wait()
                    j0 = t * 16
                    rv = ssr_v[pl.ds(sbase + j0, 16)]
                    nv = ssn_v[pl.ds(sbase + j0, 16)]
                    vm = (j0 + iota) < cnt
                    rloc = (rv - ss_base) & (_SS_ROWS - 1)
                    lane = rloc & 127
                    wrow = (rloc >> 7) << 6
                    nv_s = jnp.where(vm, nv, 0)
                    pos64 = lax.rem(nv_s, L) * D
                    nst[pl.ds(0, 16)] = jnp.where(vm, nv, dump16)

                    @pl.loop(0, D, unroll=8)
                    def _feat(c):
                        tok = plsc.load_gather(wbuf, [wrow + c, lane])
                        pv = plsc.load_gather(pe_v, [pos64 + c])
                        plsc.store_scatter(lbuf, [iota, iota * 0 + c],
                                           tok + pv)
                    pltpu.async_copy(lbuf, out_hbm.at[nst], lsem)

        start_win(ss + 2, wbuf, wsem)

    @pl.loop(0, 8)
    def _octloop(o):
        fill_ss_lists(o)

        @pl.loop(0, 4)
        def _sspair(sp):
            for b in range(2):
                si = sp * 2 + b
                ss = o * 8 + si

                @pl.when(ss < nss)
                def _run():
                    do_subshard(ss, si, wins[b][0], wins[b][1])

    for lbuf, nst, lsem in lines:
        pltpu.make_async_copy(out_hbm.at[pl.ds(0, 16)], lbuf, lsem).wait()


@jax.jit
def _sweep(tT, xi, pe, tailT):
    f = pl.kernel(
        _sweep_body,
        mesh=plsc.VectorSubcoreMesh(core_axis_name="c", subcore_axis_name="s"),
        out_type=jax.ShapeDtypeStruct((_N + 1, 128), jnp.float32),
        scratch_types=[
            pltpu.VMEM((_SS_ROWS // 2, 128), jnp.float32),
            pltpu.VMEM((_SS_ROWS // 2, 128), jnp.float32),
            pltpu.VMEM((_CHUNK,), jnp.int32),
            pltpu.VMEM((_CHUNK,), jnp.int32),
            pltpu.VMEM((4 * _REG_CAP,), jnp.int32),
            pltpu.VMEM((4 * _REG_CAP,), jnp.int32),
            pltpu.VMEM((8 * _OCT_CAP,), jnp.int32),
            pltpu.VMEM((8 * _OCT_CAP,), jnp.int32),
            pltpu.VMEM((8 * _SS_CAP,), jnp.int32),
            pltpu.VMEM((8 * _SS_CAP,), jnp.int32),
            pltpu.VMEM((L * D,), jnp.float32),
            pltpu.VMEM((16, 128), jnp.float32),
            pltpu.VMEM((16, 128), jnp.float32),
            pltpu.VMEM((16, 128), jnp.float32),
            pltpu.VMEM((16, 128), jnp.float32),
            pltpu.VMEM((16, 128), jnp.float32),
            pltpu.VMEM((16,), jnp.int32),
            pltpu.VMEM((16,), jnp.int32),
            pltpu.VMEM((16,), jnp.int32),
            pltpu.VMEM((16,), jnp.int32),
            pltpu.VMEM((16,), jnp.int32),
            pltpu.SMEM((8,), jnp.int32),
            pltpu.SMEM((8,), jnp.int32),
            pltpu.SMEM((4,), jnp.int32),
            pltpu.SemaphoreType.DMA,
            pltpu.SemaphoreType.DMA,
            pltpu.SemaphoreType.DMA,
            pltpu.SemaphoreType.DMA,
            pltpu.SemaphoreType.DMA,
            pltpu.SemaphoreType.DMA,
            pltpu.SemaphoreType.DMA,
            pltpu.SemaphoreType.DMA,
            pltpu.SemaphoreType.DMA,
        ],
        compiler_params=pltpu.CompilerParams(
            use_tc_tiling_on_sc=True, needs_layout_passes=False
        ),
    )
    return f(tT, xi, pe, tailT)


def kernel(x, table):
    tT = table.T
    xi = x.astype(jnp.int32).reshape(_N)
    tailT = jnp.pad(tT[:, VOCAB - 64:], ((0, 0), (0, 64)))
    padded = _sweep(tT, xi, jnp.asarray(_PE_FLAT), tailT)
    return padded[:_N, :64].reshape(B, L, D)
